# baseline (speedup 1.0000x reference)
import functools

import jax
import jax.numpy as jnp
from jax import lax
from jax.experimental import pallas as pl
from jax.experimental.pallas import tpu as pltpu
from jax.experimental.pallas import tpu_sc as plsc

BATCH = 16384
_INFO = plsc.get_sparse_core_info()
_NS, _L = _INFO.num_subcores, _INFO.num_lanes
_NC = 1
_NW = _NC * _NS
_BPW = BATCH // _NW
_NCH = 4
_CL = _BPW // _NCH


def _mean_model_sc(uid_hbm, mid_hbm, utab_hbm, mtab_hbm, mean_hbm, out_hbm,
                   uid_v, mid_v, ub_v, mb_v, zidx_v, mean_v,
                   sems_u, sems_m, sem_mn):
    wid = lax.axis_index("s") * _NC + lax.axis_index("c")
    base = wid * _BPW

    ci_u = pltpu.async_copy(uid_hbm.at[pl.ds(base, _BPW)], uid_v, sems_u[0])
    ci_m = pltpu.async_copy(mid_hbm.at[pl.ds(base, _BPW)], mid_v, sems_m[0])
    zidx_v[...] = jnp.zeros((_L,), jnp.int32)
    c_mn = pltpu.async_copy(mean_hbm.at[zidx_v], mean_v, sem_mn)

    ci_u.wait()
    ci_m.wait()
    cus, cms = [], []
    for k in range(_NCH):
        cus.append(pltpu.async_copy(utab_hbm.at[uid_v.at[pl.ds(k * _CL, _CL)]],
                                    ub_v.at[pl.ds(k * _CL, _CL)], sems_u[k]))
        cms.append(pltpu.async_copy(mtab_hbm.at[mid_v.at[pl.ds(k * _CL, _CL)]],
                                    mb_v.at[pl.ds(k * _CL, _CL)], sems_m[k]))
    c_mn.wait()
    mean_vec = mean_v[...]

    def step(i, carry):
        sl = pl.ds(i * _L, _L)
        ub_v[sl] = ub_v[sl] + mb_v[sl] + mean_vec
        return carry

    cws = []
    for k in range(_NCH):
        cus[k].wait()
        cms[k].wait()
        lax.fori_loop(k * _CL // _L, (k + 1) * _CL // _L, step, 0)
        cws.append(pltpu.async_copy(ub_v.at[pl.ds(k * _CL, _CL)],
                                    out_hbm.at[pl.ds(base + k * _CL, _CL)],
                                    sem_mn))
    for cw in cws:
        cw.wait()


@jax.jit
def _run(uid, mid, utab, mtab, mean1):
    mesh = plsc.VectorSubcoreMesh(core_axis_name="c", subcore_axis_name="s",
                                  num_cores=_NC)
    k = functools.partial(
        pl.kernel,
        mesh=mesh,
        out_type=jax.ShapeDtypeStruct((BATCH,), jnp.float32),
        scratch_types=[
            pltpu.VMEM((_BPW,), jnp.int32),
            pltpu.VMEM((_BPW,), jnp.int32),
            pltpu.VMEM((_BPW,), jnp.float32),
            pltpu.VMEM((_BPW,), jnp.float32),
            pltpu.VMEM((_L,), jnp.int32),
            pltpu.VMEM((_L,), jnp.float32),
            [pltpu.SemaphoreType.DMA] * _NCH,
            [pltpu.SemaphoreType.DMA] * _NCH,
            pltpu.SemaphoreType.DMA,
        ],
    )(_mean_model_sc)
    return k(uid, mid, utab, mtab, mean1)


def kernel(userId, movieId, user_table, movie_table, mean):
    uid = userId.astype(jnp.int32)
    mid = movieId.astype(jnp.int32)
    mean1 = jnp.asarray(mean, jnp.float32).reshape((1,))
    return _run(uid, mid, user_table, movie_table, mean1)


# speedup vs baseline: 1.0055x; 1.0055x over previous
import functools

import jax
import jax.numpy as jnp
from jax import lax
from jax.experimental import pallas as pl
from jax.experimental.pallas import tpu as pltpu
from jax.experimental.pallas import tpu_sc as plsc

BATCH = 16384
_INFO = plsc.get_sparse_core_info()
_NS, _L = _INFO.num_subcores, _INFO.num_lanes
_NC = 1
_NW = _NC * _NS
_BPW = BATCH // _NW
_C0 = 768
_C1 = _BPW - _C0


def _mean_model_sc(uid_hbm, mid_hbm, utab_hbm, mtab_hbm, mean_hbm, out_hbm,
                   uid_v, mid_v, ub_v, mb_v, zidx_v, mean_v,
                   sem_u, sem_m, sem_u1, sem_m1, sem_mn):
    wid = lax.axis_index("s") * _NC + lax.axis_index("c")
    base = wid * _BPW

    ci_u = pltpu.async_copy(uid_hbm.at[pl.ds(base, _BPW)], uid_v, sem_u)
    ci_m = pltpu.async_copy(mid_hbm.at[pl.ds(base, _BPW)], mid_v, sem_m)
    zidx_v[...] = jnp.zeros((_L,), jnp.int32)
    c_mn = pltpu.async_copy(mean_hbm.at[zidx_v], mean_v, sem_mn)

    ci_u.wait()
    cu0 = pltpu.async_copy(utab_hbm.at[uid_v.at[pl.ds(0, _C0)]],
                           ub_v.at[pl.ds(0, _C0)], sem_u)
    cu1 = pltpu.async_copy(utab_hbm.at[uid_v.at[pl.ds(_C0, _C1)]],
                           ub_v.at[pl.ds(_C0, _C1)], sem_u1)
    ci_m.wait()
    cm0 = pltpu.async_copy(mtab_hbm.at[mid_v.at[pl.ds(0, _C0)]],
                           mb_v.at[pl.ds(0, _C0)], sem_m)
    cm1 = pltpu.async_copy(mtab_hbm.at[mid_v.at[pl.ds(_C0, _C1)]],
                           mb_v.at[pl.ds(_C0, _C1)], sem_m1)
    c_mn.wait()
    mean_vec = mean_v[...]

    def step(i, carry):
        sl = pl.ds(i * _L, _L)
        ub_v[sl] = ub_v[sl] + mb_v[sl] + mean_vec
        return carry

    cu0.wait()
    cm0.wait()
    lax.fori_loop(0, _C0 // _L, step, 0)
    cw0 = pltpu.async_copy(ub_v.at[pl.ds(0, _C0)],
                           out_hbm.at[pl.ds(base, _C0)], sem_mn)
    cu1.wait()
    cm1.wait()
    lax.fori_loop(_C0 // _L, _BPW // _L, step, 0)
    cw1 = pltpu.async_copy(ub_v.at[pl.ds(_C0, _C1)],
                           out_hbm.at[pl.ds(base + _C0, _C1)], sem_u)
    cw0.wait()
    cw1.wait()


@jax.jit
def _run(uid, mid, utab, mtab, mean1):
    mesh = plsc.VectorSubcoreMesh(core_axis_name="c", subcore_axis_name="s",
                                  num_cores=_NC)
    k = functools.partial(
        pl.kernel,
        mesh=mesh,
        out_type=jax.ShapeDtypeStruct((BATCH,), jnp.float32),
        scratch_types=[
            pltpu.VMEM((_BPW,), jnp.int32),
            pltpu.VMEM((_BPW,), jnp.int32),
            pltpu.VMEM((_BPW,), jnp.float32),
            pltpu.VMEM((_BPW,), jnp.float32),
            pltpu.VMEM((_L,), jnp.int32),
            pltpu.VMEM((_L,), jnp.float32),
            pltpu.SemaphoreType.DMA,
            pltpu.SemaphoreType.DMA,
            pltpu.SemaphoreType.DMA,
            pltpu.SemaphoreType.DMA,
            pltpu.SemaphoreType.DMA,
        ],
    )(_mean_model_sc)
    return k(uid, mid, utab, mtab, mean1)


def kernel(userId, movieId, user_table, movie_table, mean):
    uid = userId.astype(jnp.int32)
    mid = movieId.astype(jnp.int32)
    mean1 = jnp.asarray(mean, jnp.float32).reshape((1,))
    return _run(uid, mid, user_table, movie_table, mean1)


# speedup vs baseline: 1.0169x; 1.0114x over previous
import functools

import jax
import jax.numpy as jnp
from jax import lax
from jax.experimental import pallas as pl
from jax.experimental.pallas import tpu as pltpu
from jax.experimental.pallas import tpu_sc as plsc

BATCH = 16384
_INFO = plsc.get_sparse_core_info()
_NS, _L = _INFO.num_subcores, _INFO.num_lanes
_NC = 1
_NW = _NC * _NS
_BPW = BATCH // _NW
_C0 = 512
_C1 = _BPW - _C0


def _mean_model_sc(uid_hbm, mid_hbm, utab_hbm, mtab_hbm, mean_hbm, out_hbm,
                   uid_v, mid_v, ub_v, mb_v, zidx_v, mean_v,
                   sem_u, sem_m, sem_u1, sem_m1, sem_mn):
    wid = lax.axis_index("s") * _NC + lax.axis_index("c")
    base = wid * _BPW

    ci_u = pltpu.async_copy(uid_hbm.at[pl.ds(base, _BPW)], uid_v, sem_u)
    ci_m = pltpu.async_copy(mid_hbm.at[pl.ds(base, _BPW)], mid_v, sem_m)
    zidx_v[...] = jnp.zeros((_L,), jnp.int32)
    c_mn = pltpu.async_copy(mean_hbm.at[zidx_v], mean_v, sem_mn)

    ci_u.wait()
    cu0 = pltpu.async_copy(utab_hbm.at[uid_v.at[pl.ds(0, _C0)]],
                           ub_v.at[pl.ds(0, _C0)], sem_u)
    cu1 = pltpu.async_copy(utab_hbm.at[uid_v.at[pl.ds(_C0, _C1)]],
                           ub_v.at[pl.ds(_C0, _C1)], sem_u1)
    ci_m.wait()
    cm0 = pltpu.async_copy(mtab_hbm.at[mid_v.at[pl.ds(0, _C0)]],
                           mb_v.at[pl.ds(0, _C0)], sem_m)
    cm1 = pltpu.async_copy(mtab_hbm.at[mid_v.at[pl.ds(_C0, _C1)]],
                           mb_v.at[pl.ds(_C0, _C1)], sem_m1)
    c_mn.wait()
    mean_vec = mean_v[...]

    def step(i, carry):
        sl = pl.ds(i * _L, _L)
        ub_v[sl] = ub_v[sl] + mb_v[sl] + mean_vec
        return carry

    cu0.wait()
    cm0.wait()
    lax.fori_loop(0, _C0 // _L, step, 0)
    cw0 = pltpu.async_copy(ub_v.at[pl.ds(0, _C0)],
                           out_hbm.at[pl.ds(base, _C0)], sem_mn)
    cu1.wait()
    cm1.wait()
    lax.fori_loop(_C0 // _L, _BPW // _L, step, 0)
    cw1 = pltpu.async_copy(ub_v.at[pl.ds(_C0, _C1)],
                           out_hbm.at[pl.ds(base + _C0, _C1)], sem_u)
    cw0.wait()
    cw1.wait()


@jax.jit
def _run(uid, mid, utab, mtab, mean1):
    mesh = plsc.VectorSubcoreMesh(core_axis_name="c", subcore_axis_name="s",
                                  num_cores=_NC)
    k = functools.partial(
        pl.kernel,
        mesh=mesh,
        out_type=jax.ShapeDtypeStruct((BATCH,), jnp.float32),
        scratch_types=[
            pltpu.VMEM((_BPW,), jnp.int32),
            pltpu.VMEM((_BPW,), jnp.int32),
            pltpu.VMEM((_BPW,), jnp.float32),
            pltpu.VMEM((_BPW,), jnp.float32),
            pltpu.VMEM((_L,), jnp.int32),
            pltpu.VMEM((_L,), jnp.float32),
            pltpu.SemaphoreType.DMA,
            pltpu.SemaphoreType.DMA,
            pltpu.SemaphoreType.DMA,
            pltpu.SemaphoreType.DMA,
            pltpu.SemaphoreType.DMA,
        ],
    )(_mean_model_sc)
    return k(uid, mid, utab, mtab, mean1)


def kernel(userId, movieId, user_table, movie_table, mean):
    uid = userId.astype(jnp.int32)
    mid = movieId.astype(jnp.int32)
    mean1 = jnp.asarray(mean, jnp.float32).reshape((1,))
    return _run(uid, mid, user_table, movie_table, mean1)
